# baseline (speedup 1.0000x reference)
import functools
import math

import jax
import jax.numpy as jnp
from jax import lax
from jax.experimental import pallas as pl
from jax.experimental.pallas import tpu as pltpu
from jax.experimental.pallas import tpu_sc as plsc

_N = 10000
_E = 320000
_H = 128
_G = 50
_F = 128
_CUTOFF = 10.0

_BE = 2560
_NBLK = _E // _BE

_NC = 2
_NS = 16
_NW = _NC * _NS
_CH = 64
_NCH = 160
_NPCH = _NCH * _NW
_EP = _NPCH * _CH
_ACC = 10240
_STRIPE = _ACC // _NS

_LOG2 = math.log(2.0)


def _ssp(v):
    return jnp.maximum(v, 0.0) + jnp.log1p(jnp.exp(-jnp.abs(v))) - _LOG2


def _dot_t(a, b):
    return lax.dot_general(a, b, (((1,), (1,)), ((), ())),
                           preferred_element_type=jnp.float32)


def _filter_body(basist_ref, e_ref, wf1_ref, bf1c_ref, wf2a_ref, w_ref):
    v = lax.dot_general(wf1_ref[...], basist_ref[...], (((1,), (0,)), ((), ())),
                        precision=lax.Precision.HIGHEST,
                        preferred_element_type=jnp.float32) + bf1c_ref[...]
    c = 0.25 * (jnp.cos(e_ref[0] * (math.pi / _CUTOFF)) + 1.0)
    h_aug = jnp.concatenate([_ssp(v) * c, c, jnp.zeros((7, _BE), jnp.float32)],
                            axis=0)
    w_ref[...] = lax.dot_general(h_aug, wf2a_ref[...], (((0,), (1,)), ((), ())),
                                 precision=lax.Precision.HIGHEST,
                                 preferred_element_type=jnp.float32)


_filter_call = pl.pallas_call(
    _filter_body,
    grid=(_NBLK,),
    in_specs=[
        pl.BlockSpec((_G, _BE), lambda i: (0, i)),
        pl.BlockSpec((1, 1, _BE), lambda i: (i, 0, 0)),
        pl.BlockSpec((_F, _G), lambda i: (0, 0)),
        pl.BlockSpec((_F, 1), lambda i: (0, 0)),
        pl.BlockSpec((_F, _F + 8), lambda i: (0, 0)),
    ],
    out_specs=pl.BlockSpec((_BE, _F), lambda i: (i, 0)),
    out_shape=jax.ShapeDtypeStruct((_E, _F), jnp.float32),
)


def _lin1_body(x_ref, wl1_ref, o_ref):
    o_ref[...] = _dot_t(x_ref[...], wl1_ref[...])


_lin1_call = pl.pallas_call(
    _lin1_body,
    grid=(5,),
    in_specs=[
        pl.BlockSpec((_N // 5, _H), lambda i: (i, 0)),
        pl.BlockSpec((_F, _H), lambda i: (0, 0)),
    ],
    out_specs=pl.BlockSpec((_N // 5, _F), lambda i: (i, 0)),
    out_shape=jax.ShapeDtypeStruct((_N, _F), jnp.float32),
)


_sc_mesh = plsc.VectorSubcoreMesh(core_axis_name="c", subcore_axis_name="s")


@functools.partial(
    pl.kernel,
    mesh=_sc_mesh,
    out_type=jax.ShapeDtypeStruct((_NC, _ACC, _H), jnp.float32),
    scratch_types=[
        pltpu.VMEM((4, _CH), jnp.int32),
        pltpu.VMEM((4, _CH), jnp.int32),
        pltpu.VMEM((2, _CH, _H), jnp.float32),
        pltpu.VMEM((2, _CH, _H), jnp.float32),
        pltpu.VMEM_SHARED((_ACC, _H), jnp.float32),
        pltpu.SemaphoreType.DMA,
        pltpu.SemaphoreType.DMA,
        pltpu.SemaphoreType.DMA,
        pltpu.SemaphoreType.DMA,
        pltpu.SemaphoreType.DMA,
        pltpu.SemaphoreType.DMA,
        pltpu.SemaphoreType.DMA,
        pltpu.SemaphoreType.DMA,
    ],
)
def _sc_aggregate(x1_hbm, w_hbm, src_hbm, dst_hbm, out_hbm,
                  sidx, didx, wbuf, rbuf, acc,
                  i0s, i1s, i2s, i3s, g0s, g1s, w0s, w1s):
    cid = lax.axis_index("c")
    sid = lax.axis_index("s")
    wid = cid * _NS + sid
    isem = (i0s, i1s, i2s, i3s)
    gsem = (g0s, g1s)
    wsem = (w0s, w1s)

    def _zrow(e, carry):
        for j in range(_H // 16):
            rbuf[0, e, pl.ds(j * 16, 16)] = jnp.zeros((16,), jnp.float32)
        return carry

    lax.fori_loop(0, _CH, _zrow, 0)

    def _zcopy(k, carry):
        pltpu.sync_copy(rbuf.at[0], acc.at[pl.ds(sid * _STRIPE + k * _CH, _CH)])
        return carry

    lax.fori_loop(0, _STRIPE // _CH, _zcopy, 0)
    plsc.subcore_barrier()

    base0 = wid * _NCH

    def _iload(i, a):
        e0 = (base0 + i) * _CH
        pltpu.async_copy(src_hbm.at[pl.ds(e0, _CH)], sidx.at[a], isem[a])
        pltpu.async_copy(dst_hbm.at[pl.ds(e0, _CH)], didx.at[a], isem[a])

    def _iwait(i, a):
        e0 = (base0 + i) * _CH
        pltpu.make_async_copy(src_hbm.at[pl.ds(e0, _CH)], sidx.at[a],
                              isem[a]).wait()
        pltpu.make_async_copy(dst_hbm.at[pl.ds(e0, _CH)], didx.at[a],
                              isem[a]).wait()

    def _wslab(i):
        return jnp.minimum((base0 + i) * _CH, _E - _CH)

    def _start(i, a, b):
        pltpu.async_copy(w_hbm.at[pl.ds(_wslab(i), _CH)], wbuf.at[b], wsem[b])
        pltpu.async_copy(x1_hbm.at[sidx.at[a]], rbuf.at[b], gsem[b])

    def _finish(i, a, b):
        pltpu.make_async_copy(w_hbm.at[pl.ds(_wslab(i), _CH)], wbuf.at[b],
                              wsem[b]).wait()
        pltpu.make_async_copy(x1_hbm.at[sidx.at[a]], rbuf.at[b],
                              gsem[b]).wait()

        def _mul(e, inner):
            for j in range(_H // 16):
                sl = pl.ds(j * 16, 16)
                rbuf[b, e, sl] = rbuf[b, e, sl] * wbuf[b, e, sl]
            return inner

        lax.fori_loop(0, _CH, _mul, 0)
        pltpu.sync_copy(rbuf.at[b], acc.at[didx.at[a]], add=True)

    _iload(0, 0)
    _iload(1, 1)
    _iwait(0, 0)
    _start(0, 0, 0)

    def _group(g4, carry):
        for k in range(4):
            i = g4 * 4 + k

            @pl.when(i < _NCH - 1)
            def _adv():
                _iwait(i + 1, (k + 1) % 4)
                _start(i + 1, (k + 1) % 4, (k + 1) % 2)

            @pl.when(i < _NCH - 2)
            def _pref():
                _iload(i + 2, (k + 2) % 4)

            _finish(i, k % 4, k % 2)
        return carry

    lax.fori_loop(0, _NCH // 4, _group, 0)

    plsc.subcore_barrier()
    pltpu.sync_copy(acc.at[pl.ds(sid * _STRIPE, _STRIPE)],
                    out_hbm.at[cid, pl.ds(sid * _STRIPE, _STRIPE)])


def _out_body(p_ref, wl2_ref, bl2_ref, wl3_ref, bl3_ref, o_ref):
    agg = p_ref[0] + p_ref[1]
    x2 = _dot_t(agg, wl2_ref[...]) + bl2_ref[...]
    o_ref[...] = _dot_t(_ssp(x2), wl3_ref[...]) + bl3_ref[...]


_out_call = pl.pallas_call(
    _out_body,
    grid=(5,),
    in_specs=[
        pl.BlockSpec((_NC, _N // 5, _H), lambda i: (0, i, 0)),
        pl.BlockSpec((_H, _F), lambda i: (0, 0)),
        pl.BlockSpec((1, _H), lambda i: (0, 0)),
        pl.BlockSpec((_H, _H), lambda i: (0, 0)),
        pl.BlockSpec((1, _H), lambda i: (0, 0)),
    ],
    out_specs=pl.BlockSpec((_N // 5, _H), lambda i: (i, 0)),
    out_shape=jax.ShapeDtypeStruct((_N, _H), jnp.float32),
)


def kernel(x, ji_pairs, e_ji, e_ji_basis, Wf1, bf1, Wf2, bf2,
           Wl1, Wl2, bl2, Wl3, bl3):
    npad = _EP - _E
    fill_src = jnp.arange(npad, dtype=jnp.int32) % _N
    fill_dst = _N + jnp.arange(npad, dtype=jnp.int32) % (_ACC - _N)
    src = jnp.concatenate([ji_pairs[0].astype(jnp.int32), fill_src])
    dst = jnp.concatenate([ji_pairs[1].astype(jnp.int32), fill_dst])
    e3d = e_ji.reshape(_NBLK, 1, _BE)
    wf2_aug = jnp.concatenate([Wf2, bf2[:, None],
                               jnp.zeros((_F, 7), jnp.float32)], axis=1)

    w_edge = _filter_call(e_ji_basis.T, e3d, Wf1, bf1[:, None], wf2_aug)
    x1 = _lin1_call(x, Wl1)
    partial = _sc_aggregate(x1, w_edge, src, dst)
    out = _out_call(partial, Wl2, bl2[None, :], Wl3, bl3[None, :])
    return out


# speedup vs baseline: 1.3171x; 1.3171x over previous
import functools
import math

import jax
import jax.numpy as jnp
from jax import lax
from jax.experimental import pallas as pl
from jax.experimental.pallas import tpu as pltpu
from jax.experimental.pallas import tpu_sc as plsc

_N = 10000
_E = 320000
_H = 128
_G = 50
_F = 128
_CUTOFF = 10.0

_BE = 2560

_NC = 2
_NS = 16
_NW = _NC * _NS
_CH = 64
_NCH = 80
_EH = _NW * _NCH * _CH
_EB = _E - _EH
_ACC = 10240
_STRIPE = _ACC // _NS

_LOG2 = math.log(2.0)


def _ssp(v):
    return jnp.maximum(v, 0.0) + jnp.log1p(jnp.exp(-jnp.abs(v))) - _LOG2


def _dot_t(a, b):
    return lax.dot_general(a, b, (((1,), (1,)), ((), ())),
                           preferred_element_type=jnp.float32)


def _filter_body(basis_ref, e_ref, wf1_ref, bf1_ref, wf2_ref, bf2_ref,
                 w_ref, c_ref):
    h = _ssp(_dot_t(basis_ref[...], wf1_ref[...]) + bf1_ref[...])
    w_ref[...] = _dot_t(h, wf2_ref[...]) + bf2_ref[...]
    c_ref[...] = 0.25 * (jnp.cos(e_ref[...] * (math.pi / _CUTOFF)) + 1.0)


def _make_filter(nblk):
    return pl.pallas_call(
        _filter_body,
        grid=(nblk,),
        in_specs=[
            pl.BlockSpec((_BE, _G), lambda i: (i, 0)),
            pl.BlockSpec((1, 1, _BE), lambda i: (i, 0, 0)),
            pl.BlockSpec((_F, _G), lambda i: (0, 0)),
            pl.BlockSpec((1, _F), lambda i: (0, 0)),
            pl.BlockSpec((_F, _F), lambda i: (0, 0)),
            pl.BlockSpec((1, _F), lambda i: (0, 0)),
        ],
        out_specs=[
            pl.BlockSpec((_BE, _F), lambda i: (i, 0)),
            pl.BlockSpec((1, 1, _BE), lambda i: (i, 0, 0)),
        ],
        out_shape=[
            jax.ShapeDtypeStruct((nblk * _BE, _F), jnp.float32),
            jax.ShapeDtypeStruct((nblk, 1, _BE), jnp.float32),
        ],
    )


_filter_a = _make_filter(_EH // _BE)
_filter_b = _make_filter(_EB // _BE)


def _lin1_body(x_ref, wl1_ref, o_ref):
    o_ref[...] = _dot_t(x_ref[...], wl1_ref[...])


_lin1_call = pl.pallas_call(
    _lin1_body,
    grid=(5,),
    in_specs=[
        pl.BlockSpec((_N // 5, _H), lambda i: (i, 0)),
        pl.BlockSpec((_F, _H), lambda i: (0, 0)),
    ],
    out_specs=pl.BlockSpec((_N // 5, _F), lambda i: (i, 0)),
    out_shape=jax.ShapeDtypeStruct((_N, _F), jnp.float32),
)


_sc_mesh = plsc.VectorSubcoreMesh(core_axis_name="c", subcore_axis_name="s")


def _make_sc(e_real):
    @functools.partial(
        pl.kernel,
        mesh=_sc_mesh,
        out_type=jax.ShapeDtypeStruct((_NC, _ACC, _H), jnp.float32),
        scratch_types=[
            pltpu.VMEM((4, _CH), jnp.int32),
            pltpu.VMEM((4, _CH), jnp.int32),
            pltpu.VMEM((4, _CH), jnp.float32),
            pltpu.VMEM((2, _CH, _H), jnp.float32),
            pltpu.VMEM((2, _CH, _H), jnp.float32),
            pltpu.VMEM_SHARED((_ACC, _H), jnp.float32),
            pltpu.SemaphoreType.DMA,
            pltpu.SemaphoreType.DMA,
            pltpu.SemaphoreType.DMA,
            pltpu.SemaphoreType.DMA,
            pltpu.SemaphoreType.DMA,
            pltpu.SemaphoreType.DMA,
            pltpu.SemaphoreType.DMA,
            pltpu.SemaphoreType.DMA,
        ],
    )
    def _sc_aggregate(x1_hbm, w_hbm, c_hbm, src_hbm, dst_hbm, out_hbm,
                      sidx, didx, cbuf, wbuf, rbuf, acc,
                      i0s, i1s, i2s, i3s, g0s, g1s, w0s, w1s):
        cid = lax.axis_index("c")
        sid = lax.axis_index("s")
        wid = cid * _NS + sid
        isem = (i0s, i1s, i2s, i3s)
        gsem = (g0s, g1s)
        wsem = (w0s, w1s)

        def _zrow(e, carry):
            for j in range(_H // 16):
                rbuf[0, e, pl.ds(j * 16, 16)] = jnp.zeros((16,), jnp.float32)
            return carry

        lax.fori_loop(0, _CH, _zrow, 0)

        def _zcopy(k, carry):
            pltpu.sync_copy(rbuf.at[0],
                            acc.at[pl.ds(sid * _STRIPE + k * _CH, _CH)])
            return carry

        lax.fori_loop(0, _STRIPE // _CH, _zcopy, 0)
        plsc.subcore_barrier()

        base0 = wid * _NCH

        def _iload(i, a):
            e0 = (base0 + i) * _CH
            pltpu.async_copy(src_hbm.at[pl.ds(e0, _CH)], sidx.at[a], isem[a])
            pltpu.async_copy(dst_hbm.at[pl.ds(e0, _CH)], didx.at[a], isem[a])
            pltpu.async_copy(c_hbm.at[pl.ds(e0, _CH)], cbuf.at[a], isem[a])

        def _iwait(i, a):
            e0 = (base0 + i) * _CH
            pltpu.make_async_copy(src_hbm.at[pl.ds(e0, _CH)], sidx.at[a],
                                  isem[a]).wait()
            pltpu.make_async_copy(dst_hbm.at[pl.ds(e0, _CH)], didx.at[a],
                                  isem[a]).wait()
            pltpu.make_async_copy(c_hbm.at[pl.ds(e0, _CH)], cbuf.at[a],
                                  isem[a]).wait()

        def _wslab(i):
            return jnp.minimum((base0 + i) * _CH, e_real - _CH)

        def _start(i, a, b):
            pltpu.async_copy(w_hbm.at[pl.ds(_wslab(i), _CH)], wbuf.at[b],
                             wsem[b])
            pltpu.async_copy(x1_hbm.at[sidx.at[a]], rbuf.at[b], gsem[b])

        def _finish(i, a, b):
            pltpu.make_async_copy(w_hbm.at[pl.ds(_wslab(i), _CH)], wbuf.at[b],
                                  wsem[b]).wait()
            pltpu.make_async_copy(x1_hbm.at[sidx.at[a]], rbuf.at[b],
                                  gsem[b]).wait()

            def _mul(g, inner):
                cv16 = cbuf[a, pl.ds(g * 16, 16)]
                for k in range(16):
                    e = g * 16 + k
                    cv = cv16[k]
                    for j in range(_H // 16):
                        sl = pl.ds(j * 16, 16)
                        rbuf[b, e, sl] = rbuf[b, e, sl] * (wbuf[b, e, sl] * cv)
                return inner

            lax.fori_loop(0, _CH // 16, _mul, 0)
            pltpu.sync_copy(rbuf.at[b], acc.at[didx.at[a]], add=True)

        _iload(0, 0)
        _iload(1, 1)
        _iwait(0, 0)
        _start(0, 0, 0)

        def _group(g4, carry):
            for k in range(4):
                i = g4 * 4 + k

                @pl.when(i < _NCH - 1)
                def _adv():
                    _iwait(i + 1, (k + 1) % 4)
                    _start(i + 1, (k + 1) % 4, (k + 1) % 2)

                @pl.when(i < _NCH - 2)
                def _pref():
                    _iload(i + 2, (k + 2) % 4)

                _finish(i, k % 4, k % 2)
            return carry

        lax.fori_loop(0, _NCH // 4, _group, 0)

        plsc.subcore_barrier()
        pltpu.sync_copy(acc.at[pl.ds(sid * _STRIPE, _STRIPE)],
                        out_hbm.at[cid, pl.ds(sid * _STRIPE, _STRIPE)])

    return _sc_aggregate


_sc_a = _make_sc(_EH)
_sc_b = _make_sc(_EB)


def _out_body(pa_ref, pb_ref, wl2_ref, bl2_ref, wl3_ref, bl3_ref, o_ref):
    agg = (pa_ref[0] + pa_ref[1]) + (pb_ref[0] + pb_ref[1])
    x2 = _dot_t(agg, wl2_ref[...]) + bl2_ref[...]
    o_ref[...] = _dot_t(_ssp(x2), wl3_ref[...]) + bl3_ref[...]


_out_call = pl.pallas_call(
    _out_body,
    grid=(5,),
    in_specs=[
        pl.BlockSpec((_NC, _N // 5, _H), lambda i: (0, i, 0)),
        pl.BlockSpec((_NC, _N // 5, _H), lambda i: (0, i, 0)),
        pl.BlockSpec((_H, _F), lambda i: (0, 0)),
        pl.BlockSpec((1, _H), lambda i: (0, 0)),
        pl.BlockSpec((_H, _H), lambda i: (0, 0)),
        pl.BlockSpec((1, _H), lambda i: (0, 0)),
    ],
    out_specs=pl.BlockSpec((_N // 5, _H), lambda i: (i, 0)),
    out_shape=jax.ShapeDtypeStruct((_N, _H), jnp.float32),
)


def kernel(x, ji_pairs, e_ji, e_ji_basis, Wf1, bf1, Wf2, bf2,
           Wl1, Wl2, bl2, Wl3, bl3):
    npad = _EH - _EB
    fill_src = jnp.arange(npad, dtype=jnp.int32) % _N
    fill_dst = _N + jnp.arange(npad, dtype=jnp.int32) % (_ACC - _N)
    src = ji_pairs[0].astype(jnp.int32)
    dst = ji_pairs[1].astype(jnp.int32)
    src_b = jnp.concatenate([src[_EH:], fill_src])
    dst_b = jnp.concatenate([dst[_EH:], fill_dst])

    x1 = _lin1_call(x, Wl1)
    w_a, c_a = _filter_a(e_ji_basis[:_EH], e_ji[:_EH].reshape(-1, 1, _BE),
                         Wf1, bf1[None, :], Wf2, bf2[None, :])
    p_a = _sc_a(x1, w_a, c_a.reshape(_EH), src[:_EH], dst[:_EH])
    w_b, c_b = _filter_b(e_ji_basis[_EH:], e_ji[_EH:].reshape(-1, 1, _BE),
                         Wf1, bf1[None, :], Wf2, bf2[None, :])
    c_bp = jnp.concatenate([c_b.reshape(_EB),
                            jnp.zeros(npad, jnp.float32)])
    p_b = _sc_b(x1, w_b, c_bp, src_b, dst_b)
    out = _out_call(p_a, p_b, Wl2, bl2[None, :], Wl3, bl3[None, :])
    return out


# speedup vs baseline: 1.4013x; 1.0639x over previous
import functools
import math

import jax
import jax.numpy as jnp
from jax import lax
from jax.experimental import pallas as pl
from jax.experimental.pallas import tpu as pltpu
from jax.experimental.pallas import tpu_sc as plsc

_N = 10000
_E = 320000
_H = 128
_G = 50
_F = 128
_CUTOFF = 10.0

_BE = 2560

_NC = 2
_NS = 16
_NW = _NC * _NS
_CH = 64
_NCH = 40
_EQ = _NW * _NCH * _CH
_EL = _E - 3 * _EQ
_ACC = 10240
_STRIPE = _ACC // _NS

_LOG2 = math.log(2.0)


def _ssp(v):
    return jnp.maximum(v, 0.0) + jnp.log1p(jnp.exp(-jnp.abs(v))) - _LOG2


def _dot_t(a, b):
    return lax.dot_general(a, b, (((1,), (1,)), ((), ())),
                           preferred_element_type=jnp.float32)


def _filter_body(basis_ref, e_ref, wf1_ref, bf1_ref, wf2_ref, bf2_ref,
                 w_ref, c_ref):
    h = _ssp(_dot_t(basis_ref[...], wf1_ref[...]) + bf1_ref[...])
    w_ref[...] = _dot_t(h, wf2_ref[...]) + bf2_ref[...]
    c_ref[...] = 0.25 * (jnp.cos(e_ref[...] * (math.pi / _CUTOFF)) + 1.0)


def _make_filter(nblk):
    return pl.pallas_call(
        _filter_body,
        grid=(nblk,),
        in_specs=[
            pl.BlockSpec((_BE, _G), lambda i: (i, 0)),
            pl.BlockSpec((1, 1, _BE), lambda i: (i, 0, 0)),
            pl.BlockSpec((_F, _G), lambda i: (0, 0)),
            pl.BlockSpec((1, _F), lambda i: (0, 0)),
            pl.BlockSpec((_F, _F), lambda i: (0, 0)),
            pl.BlockSpec((1, _F), lambda i: (0, 0)),
        ],
        out_specs=[
            pl.BlockSpec((_BE, _F), lambda i: (i, 0)),
            pl.BlockSpec((1, 1, _BE), lambda i: (i, 0, 0)),
        ],
        out_shape=[
            jax.ShapeDtypeStruct((nblk * _BE, _F), jnp.float32),
            jax.ShapeDtypeStruct((nblk, 1, _BE), jnp.float32),
        ],
    )


_filter_q = _make_filter(_EQ // _BE)
_filter_l = _make_filter(_EL // _BE)


def _lin1_body(x_ref, wl1_ref, o_ref):
    o_ref[...] = _dot_t(x_ref[...], wl1_ref[...])


_lin1_call = pl.pallas_call(
    _lin1_body,
    grid=(5,),
    in_specs=[
        pl.BlockSpec((_N // 5, _H), lambda i: (i, 0)),
        pl.BlockSpec((_F, _H), lambda i: (0, 0)),
    ],
    out_specs=pl.BlockSpec((_N // 5, _F), lambda i: (i, 0)),
    out_shape=jax.ShapeDtypeStruct((_N, _F), jnp.float32),
)


_sc_mesh = plsc.VectorSubcoreMesh(core_axis_name="c", subcore_axis_name="s")


def _make_sc(e_real):
    @functools.partial(
        pl.kernel,
        mesh=_sc_mesh,
        out_type=jax.ShapeDtypeStruct((_NC, _ACC, _H), jnp.float32),
        scratch_types=[
            pltpu.VMEM((4, _CH), jnp.int32),
            pltpu.VMEM((4, _CH), jnp.int32),
            pltpu.VMEM((4, _CH), jnp.float32),
            pltpu.VMEM((2, _CH, _H), jnp.float32),
            pltpu.VMEM((2, _CH, _H), jnp.float32),
            pltpu.VMEM_SHARED((_ACC, _H), jnp.float32),
            pltpu.SemaphoreType.DMA,
            pltpu.SemaphoreType.DMA,
            pltpu.SemaphoreType.DMA,
            pltpu.SemaphoreType.DMA,
            pltpu.SemaphoreType.DMA,
            pltpu.SemaphoreType.DMA,
            pltpu.SemaphoreType.DMA,
            pltpu.SemaphoreType.DMA,
        ],
    )
    def _sc_aggregate(x1_hbm, w_hbm, c_hbm, src_hbm, dst_hbm, out_hbm,
                      sidx, didx, cbuf, wbuf, rbuf, acc,
                      i0s, i1s, i2s, i3s, g0s, g1s, w0s, w1s):
        cid = lax.axis_index("c")
        sid = lax.axis_index("s")
        wid = cid * _NS + sid
        isem = (i0s, i1s, i2s, i3s)
        gsem = (g0s, g1s)
        wsem = (w0s, w1s)

        def _zrow(e, carry):
            for j in range(_H // 16):
                rbuf[0, e, pl.ds(j * 16, 16)] = jnp.zeros((16,), jnp.float32)
            return carry

        lax.fori_loop(0, _CH, _zrow, 0)

        def _zcopy(k, carry):
            pltpu.sync_copy(rbuf.at[0],
                            acc.at[pl.ds(sid * _STRIPE + k * _CH, _CH)])
            return carry

        lax.fori_loop(0, _STRIPE // _CH, _zcopy, 0)
        plsc.subcore_barrier()

        base0 = wid * _NCH

        def _iload(i, a):
            e0 = (base0 + i) * _CH
            pltpu.async_copy(src_hbm.at[pl.ds(e0, _CH)], sidx.at[a], isem[a])
            pltpu.async_copy(dst_hbm.at[pl.ds(e0, _CH)], didx.at[a], isem[a])
            pltpu.async_copy(c_hbm.at[pl.ds(e0, _CH)], cbuf.at[a], isem[a])

        def _iwait(i, a):
            e0 = (base0 + i) * _CH
            pltpu.make_async_copy(src_hbm.at[pl.ds(e0, _CH)], sidx.at[a],
                                  isem[a]).wait()
            pltpu.make_async_copy(dst_hbm.at[pl.ds(e0, _CH)], didx.at[a],
                                  isem[a]).wait()
            pltpu.make_async_copy(c_hbm.at[pl.ds(e0, _CH)], cbuf.at[a],
                                  isem[a]).wait()

        def _wslab(i):
            return jnp.minimum((base0 + i) * _CH, e_real - _CH)

        def _start(i, a, b):
            pltpu.async_copy(w_hbm.at[pl.ds(_wslab(i), _CH)], wbuf.at[b],
                             wsem[b])
            pltpu.async_copy(x1_hbm.at[sidx.at[a]], rbuf.at[b], gsem[b])

        def _finish(i, a, b):
            pltpu.make_async_copy(w_hbm.at[pl.ds(_wslab(i), _CH)], wbuf.at[b],
                                  wsem[b]).wait()
            pltpu.make_async_copy(x1_hbm.at[sidx.at[a]], rbuf.at[b],
                                  gsem[b]).wait()

            def _mul(g, inner):
                cv16 = cbuf[a, pl.ds(g * 16, 16)]
                for k in range(16):
                    e = g * 16 + k
                    cv = cv16[k]
                    for j in range(_H // 16):
                        sl = pl.ds(j * 16, 16)
                        rbuf[b, e, sl] = rbuf[b, e, sl] * (wbuf[b, e, sl] * cv)
                return inner

            lax.fori_loop(0, _CH // 16, _mul, 0)
            pltpu.sync_copy(rbuf.at[b], acc.at[didx.at[a]], add=True)

        _iload(0, 0)
        _iload(1, 1)
        _iwait(0, 0)
        _start(0, 0, 0)

        def _group(g4, carry):
            for k in range(4):
                i = g4 * 4 + k

                @pl.when(i < _NCH - 1)
                def _adv():
                    _iwait(i + 1, (k + 1) % 4)
                    _start(i + 1, (k + 1) % 4, (k + 1) % 2)

                @pl.when(i < _NCH - 2)
                def _pref():
                    _iload(i + 2, (k + 2) % 4)

                _finish(i, k % 4, k % 2)
            return carry

        lax.fori_loop(0, _NCH // 4, _group, 0)

        plsc.subcore_barrier()
        pltpu.sync_copy(acc.at[pl.ds(sid * _STRIPE, _STRIPE)],
                        out_hbm.at[cid, pl.ds(sid * _STRIPE, _STRIPE)])

    return _sc_aggregate


_sc_q = _make_sc(_EQ)
_sc_l = _make_sc(_EL)


def _out_body(p0_ref, p1_ref, p2_ref, p3_ref,
              wl2_ref, bl2_ref, wl3_ref, bl3_ref, o_ref):
    agg = ((p0_ref[0] + p0_ref[1]) + (p1_ref[0] + p1_ref[1]) +
           (p2_ref[0] + p2_ref[1]) + (p3_ref[0] + p3_ref[1]))
    x2 = _dot_t(agg, wl2_ref[...]) + bl2_ref[...]
    o_ref[...] = _dot_t(_ssp(x2), wl3_ref[...]) + bl3_ref[...]


_out_call = pl.pallas_call(
    _out_body,
    grid=(5,),
    in_specs=[
        pl.BlockSpec((_NC, _N // 5, _H), lambda i: (0, i, 0)),
        pl.BlockSpec((_NC, _N // 5, _H), lambda i: (0, i, 0)),
        pl.BlockSpec((_NC, _N // 5, _H), lambda i: (0, i, 0)),
        pl.BlockSpec((_NC, _N // 5, _H), lambda i: (0, i, 0)),
        pl.BlockSpec((_H, _F), lambda i: (0, 0)),
        pl.BlockSpec((1, _H), lambda i: (0, 0)),
        pl.BlockSpec((_H, _H), lambda i: (0, 0)),
        pl.BlockSpec((1, _H), lambda i: (0, 0)),
    ],
    out_specs=pl.BlockSpec((_N // 5, _H), lambda i: (i, 0)),
    out_shape=jax.ShapeDtypeStruct((_N, _H), jnp.float32),
)


def kernel(x, ji_pairs, e_ji, e_ji_basis, Wf1, bf1, Wf2, bf2,
           Wl1, Wl2, bl2, Wl3, bl3):
    npad = _EQ - _EL
    fill_src = jnp.arange(npad, dtype=jnp.int32) % _N
    fill_dst = _N + jnp.arange(npad, dtype=jnp.int32) % (_ACC - _N)
    src = ji_pairs[0].astype(jnp.int32)
    dst = ji_pairs[1].astype(jnp.int32)

    x1 = _lin1_call(x, Wl1)
    parts = []
    for q in range(4):
        lo = q * _EQ
        if q < 3:
            w_q, c_q = _filter_q(e_ji_basis[lo:lo + _EQ],
                                 e_ji[lo:lo + _EQ].reshape(-1, 1, _BE),
                                 Wf1, bf1[None, :], Wf2, bf2[None, :])
            parts.append(_sc_q(x1, w_q, c_q.reshape(_EQ),
                               src[lo:lo + _EQ], dst[lo:lo + _EQ]))
        else:
            w_q, c_q = _filter_l(e_ji_basis[lo:],
                                 e_ji[lo:].reshape(-1, 1, _BE),
                                 Wf1, bf1[None, :], Wf2, bf2[None, :])
            c_qp = jnp.concatenate([c_q.reshape(_EL),
                                    jnp.zeros(npad, jnp.float32)])
            parts.append(_sc_l(x1, w_q, c_qp,
                               jnp.concatenate([src[lo:], fill_src]),
                               jnp.concatenate([dst[lo:], fill_dst])))
    out = _out_call(*parts, Wl2, bl2[None, :], Wl3, bl3[None, :])
    return out
